# baseline (speedup 1.0000x reference)
import functools

import jax
import jax.numpy as jnp
from jax import lax
from jax.experimental import pallas as pl
from jax.experimental.pallas import tpu as pltpu
from jax.experimental.pallas import tpu_sc as plsc

NBINS = 64
NC = 2
NS = 16
NW = NC * NS
LANES = 16

B, C, H, W = 16, 3, 512, 512
ROWS = 64
SLAB = ROWS * W
HP_PER_W = (B * C * 2) // NW
SLABS_PER_HP = (H // 2) // ROWS
NTASK = 2 * HP_PER_W * SLABS_PER_HP
NBUF = 3
HIST = 2 * 3 * NBINS * LANES


def _sc_body(pred_hbm, target_hbm, out_hbm, buf0_v, buf1_v, buf2_v,
             hist_v, fold_v, sem0, sem1, sem2):
    wid = lax.axis_index("s") * NC + lax.axis_index("c")
    lane = lax.iota(jnp.int32, LANES)
    ones = jnp.full((LANES,), 1.0, dtype=jnp.float32)
    zeros = jnp.zeros((LANES,), dtype=jnp.float32)

    @plsc.parallel_loop(0, HIST // LANES, unroll=4)
    def _clear(i):
        hist_v[pl.ds(i * LANES, LANES)] = zeros

    bufs = (buf0_v, buf1_v, buf2_v)
    sems = (sem0, sem1, sem2)

    def _task(k):
        a, rest = divmod(k, HP_PER_W * SLABS_PER_HP)
        hp_i, slab_i = divmod(rest, SLABS_PER_HP)
        hp = wid * HP_PER_W + hp_i
        b = hp // (2 * C)
        c = (hp // 2) % C
        r = (hp % 2) * (H // 2) + slab_i * ROWS
        return a, b, c, r

    def _start(k):
        a, b, c, r = _task(k)
        ref = pred_hbm if a == 0 else target_hbm
        return pltpu.async_copy(
            ref.at[b, c, pl.ds(r, ROWS)], bufs[k % NBUF], sems[k % NBUF])

    handles = {k: _start(k) for k in range(NBUF - 1)}
    for k in range(NTASK):
        if k + NBUF - 1 < NTASK:
            handles[k + NBUF - 1] = _start(k + NBUF - 1)
        handles.pop(k).wait()

        a, _, c, _ = _task(k)
        basevec = lane + (a * 3 + c) * (NBINS * LANES)
        buf = bufs[k % NBUF]

        @plsc.parallel_loop(0, SLAB // LANES, unroll=16)
        def _vecs(j, buf=buf, basevec=basevec):
            row = j >> 5
            col = (j & 31) * LANES
            v = buf[row, pl.ds(col, LANES)]
            bits = plsc.bitcast(v + 1.0, jnp.int32)
            addr = ((bits >> 13) & 0x3F0) | basevec
            plsc.addupdate_scatter(hist_v, [addr], ones)

    lane16 = lane * LANES

    @plsc.parallel_loop(0, HIST // (LANES * LANES), unroll=2)
    def _fold(g):
        base = g * (LANES * LANES)
        acc = jnp.zeros((LANES,), dtype=jnp.float32)
        for l in range(LANES):
            acc = acc + plsc.load_gather(hist_v, [lane16 + (base + l)])
        fold_v[pl.ds(g * LANES, LANES)] = acc

    pltpu.sync_copy(fold_v, out_hbm.at[wid])


_sc_hist = functools.partial(
    pl.kernel,
    mesh=plsc.VectorSubcoreMesh(core_axis_name="c", subcore_axis_name="s"),
    out_type=jax.ShapeDtypeStruct((NW, HIST // LANES), jnp.float32),
    compiler_params=pltpu.CompilerParams(needs_layout_passes=False),
    scratch_types=[
        pltpu.VMEM((ROWS, W), jnp.float32),
        pltpu.VMEM((ROWS, W), jnp.float32),
        pltpu.VMEM((ROWS, W), jnp.float32),
        pltpu.VMEM((HIST,), jnp.float32),
        pltpu.VMEM((HIST // LANES,), jnp.float32),
        pltpu.SemaphoreType.DMA,
        pltpu.SemaphoreType.DMA,
        pltpu.SemaphoreType.DMA,
    ],
)(_sc_body)


def _tc_loss_body(x_ref, o_ref):
    x = x_ref[...].reshape(NW, 2 * 3, NBINS)
    h = jnp.sum(x, axis=0)
    s = jnp.sum(h, axis=-1, keepdims=True)
    hn = h / (s + 1e-8)
    d = jnp.abs(hn[0:3, :] - hn[3:6, :])
    o_ref[0, 0] = jnp.sum(d) / (3.0 * NBINS)


_tc_loss = pl.pallas_call(
    _tc_loss_body,
    out_shape=jax.ShapeDtypeStruct((1, 1), jnp.float32),
    out_specs=pl.BlockSpec(memory_space=pltpu.SMEM),
)


def kernel(pred, target):
    partial = _sc_hist(pred, target)
    loss = _tc_loss(partial)
    return loss.reshape(())


# speedup vs baseline: 1.0018x; 1.0018x over previous
import functools

import jax
import jax.numpy as jnp
from jax import lax
from jax.experimental import pallas as pl
from jax.experimental.pallas import tpu as pltpu
from jax.experimental.pallas import tpu_sc as plsc

NBINS = 64
NC = 2
NS = 16
NW = NC * NS
LANES = 16

B, C, H, W = 16, 3, 512, 512
ROWS = 64
SLAB = ROWS * W
HP_PER_W = (B * C * 2) // NW
SLABS_PER_HP = (H // 2) // ROWS
NTASK = 2 * HP_PER_W * SLABS_PER_HP
NBUF = 3
HIST = 2 * 3 * NBINS * LANES


def _sc_body(pred_hbm, target_hbm, out_hbm, buf0_v, buf1_v, buf2_v,
             hist_v, fold_v, sem0, sem1, sem2):
    wid = lax.axis_index("s") * NC + lax.axis_index("c")
    lane = lax.iota(jnp.int32, LANES)
    ones = jnp.full((LANES,), 1.0, dtype=jnp.float32)
    zeros = jnp.zeros((LANES,), dtype=jnp.float32)

    @plsc.parallel_loop(0, HIST // LANES, unroll=4)
    def _clear(i):
        hist_v[pl.ds(i * LANES, LANES)] = zeros

    bufs = (buf0_v, buf1_v, buf2_v)
    sems = (sem0, sem1, sem2)

    def _task(k):
        a, rest = divmod(k, HP_PER_W * SLABS_PER_HP)
        hp_i, slab_i = divmod(rest, SLABS_PER_HP)
        hp = wid * HP_PER_W + hp_i
        b = hp // (2 * C)
        c = (hp // 2) % C
        r = (hp % 2) * (H // 2) + slab_i * ROWS
        return a, b, c, r

    def _start(k):
        a, b, c, r = _task(k)
        ref = pred_hbm if a == 0 else target_hbm
        return pltpu.async_copy(
            ref.at[b, c, pl.ds(r, ROWS)], bufs[k % NBUF], sems[k % NBUF])

    handles = {k: _start(k) for k in range(NBUF - 1)}
    for k in range(NTASK):
        if k + NBUF - 1 < NTASK:
            handles[k + NBUF - 1] = _start(k + NBUF - 1)
        handles.pop(k).wait()

        a, _, c, _ = _task(k)
        basevec = lane + (a * 3 + c) * (NBINS * LANES)
        buf = bufs[k % NBUF]

        @plsc.parallel_loop(0, SLAB // LANES, unroll=8)
        def _vecs(j, buf=buf, basevec=basevec):
            row = j >> 5
            col = (j & 31) * LANES
            v = buf[row, pl.ds(col, LANES)]
            bits = plsc.bitcast(v + 1.0, jnp.int32)
            addr = ((bits >> 13) & 0x3F0) | basevec
            plsc.addupdate_scatter(hist_v, [addr], ones)

    lane16 = lane * LANES

    @plsc.parallel_loop(0, HIST // (LANES * LANES), unroll=2)
    def _fold(g):
        base = g * (LANES * LANES)
        acc = jnp.zeros((LANES,), dtype=jnp.float32)
        for l in range(LANES):
            acc = acc + plsc.load_gather(hist_v, [lane16 + (base + l)])
        fold_v[pl.ds(g * LANES, LANES)] = acc

    pltpu.sync_copy(fold_v, out_hbm.at[wid])


_sc_hist = functools.partial(
    pl.kernel,
    mesh=plsc.VectorSubcoreMesh(core_axis_name="c", subcore_axis_name="s"),
    out_type=jax.ShapeDtypeStruct((NW, HIST // LANES), jnp.float32),
    compiler_params=pltpu.CompilerParams(needs_layout_passes=False),
    scratch_types=[
        pltpu.VMEM((ROWS, W), jnp.float32),
        pltpu.VMEM((ROWS, W), jnp.float32),
        pltpu.VMEM((ROWS, W), jnp.float32),
        pltpu.VMEM((HIST,), jnp.float32),
        pltpu.VMEM((HIST // LANES,), jnp.float32),
        pltpu.SemaphoreType.DMA,
        pltpu.SemaphoreType.DMA,
        pltpu.SemaphoreType.DMA,
    ],
)(_sc_body)


def _tc_loss_body(x_ref, o_ref):
    x = x_ref[...].reshape(NW, 2 * 3, NBINS)
    h = jnp.sum(x, axis=0)
    s = jnp.sum(h, axis=-1, keepdims=True)
    hn = h / (s + 1e-8)
    d = jnp.abs(hn[0:3, :] - hn[3:6, :])
    o_ref[0, 0] = jnp.sum(d) / (3.0 * NBINS)


_tc_loss = pl.pallas_call(
    _tc_loss_body,
    out_shape=jax.ShapeDtypeStruct((1, 1), jnp.float32),
    out_specs=pl.BlockSpec(memory_space=pltpu.SMEM),
)


def kernel(pred, target):
    partial = _sc_hist(pred, target)
    loss = _tc_loss(partial)
    return loss.reshape(())
